# baseline (speedup 1.0000x reference)
import functools

import jax
import jax.numpy as jnp
from jax import lax
from jax.experimental import pallas as pl
from jax.experimental.pallas import tpu as pltpu
from jax.experimental.pallas import tpu_sc as plsc

_NC = 2
_NS = 16
_NW = _NC * _NS

_D = 128
_CG = 80
_CS = 125


def _sc_gather(ta, tb, coordflat, wrow1d, src3, dst3, e_total):
    nchunk = src3.shape[0] // _NW
    ew = nchunk * _CG
    ncrows = coordflat.shape[0]
    mesh = plsc.VectorSubcoreMesh(core_axis_name="c", subcore_axis_name="s")

    @functools.partial(
        pl.kernel,
        out_type=(jax.ShapeDtypeStruct((e_total, _D), jnp.float32),
                  jax.ShapeDtypeStruct((e_total, 16), jnp.float32)),
        mesh=mesh,
        scratch_types=[
            pltpu.VMEM((_CG,), jnp.int32),
            pltpu.VMEM((_CG,), jnp.int32),
            pltpu.VMEM((_CG,), jnp.int32),
            pltpu.VMEM((_CG,), jnp.int32),
            pltpu.VMEM((_CG, _D), jnp.float32),
            pltpu.VMEM((_CG, _D), jnp.float32),
            pltpu.VMEM((_CG, _D), jnp.float32),
            pltpu.VMEM((_CG, _D), jnp.float32),
            pltpu.VMEM((_CG, 16), jnp.float32),
            pltpu.VMEM((_CG, 16), jnp.float32),
            pltpu.VMEM((ncrows, _D), jnp.float32),
            pltpu.VMEM((_D,), jnp.float32),
            pltpu.SemaphoreType.DMA,
            pltpu.SemaphoreType.DMA,
            pltpu.SemaphoreType.DMA,
            pltpu.SemaphoreType.DMA,
        ],
        compiler_params=pltpu.CompilerParams(needs_layout_passes=False),
    )
    def gather_kernel(ta_hbm, tb_hbm, cf_hbm, wr_hbm, src_hbm, dst_hbm,
                      u_hbm, pay_hbm,
                      si0, si1, di0, di1, a0, a1, b0, b1, p0, p1,
                      coord_v, w_v, sg0, sg1, sw0, sw1):
        wid = lax.axis_index("s") * _NC + lax.axis_index("c")
        si = (si0, si1)
        di = (di0, di1)
        av = (a0, a1)
        bv = (b0, b1)
        pv = (p0, p1)
        sg = (sg0, sg1)
        sw = (sw0, sw1)
        pltpu.sync_copy(cf_hbm, coord_v)
        pltpu.sync_copy(wr_hbm, w_v)
        wtup = tuple(w_v[pl.ds(j * 16, 16)] for j in range(_D // 16))

        def load_and_gather(i, b):
            pltpu.sync_copy(src_hbm.at[wid * nchunk + i], si[b])
            pltpu.sync_copy(dst_hbm.at[wid * nchunk + i], di[b])
            return (pltpu.async_copy(ta_hbm.at[si[b]], av[b], sg[b]),
                    pltpu.async_copy(tb_hbm.at[di[b]], bv[b], sg[b]))

        def payload(b):
            one = jnp.ones((16,), jnp.int32)
            for g in range(_CG // 16):
                sl = pl.ds(g * 16, 16)
                s = si[b][sl]
                d = di[b][sl]
                rs = lax.shift_right_logical(s, 5)
                cs = lax.shift_left(lax.bitwise_and(s, 31), 2)
                rd = lax.shift_right_logical(d, 5)
                cd = lax.shift_left(lax.bitwise_and(d, 31), 2)
                xs0 = plsc.load_gather(coord_v, [rs, cs])
                xs1 = plsc.load_gather(coord_v, [rs, cs + one])
                xs2 = plsc.load_gather(coord_v, [rs, cs + 2 * one])
                xd0 = plsc.load_gather(coord_v, [rd, cd])
                xd1 = plsc.load_gather(coord_v, [rd, cd + one])
                xd2 = plsc.load_gather(coord_v, [rd, cd + 2 * one])
                erows = lax.iota(jnp.int32, 16) + g * 16
                plsc.store_scatter(pv[b], [erows, 0 * one], xs0 - xd0)
                plsc.store_scatter(pv[b], [erows, 1 * one], xs1 - xd1)
                plsc.store_scatter(pv[b], [erows, 2 * one], xs2 - xd2)
                dotm2 = (xs0 * xd0 + xs1 * xd1 + xs2 * xd2) * (-2.0)
                plsc.store_scatter(pv[b], [erows, 3 * one], dotm2)

        def fold(b):
            def ebody(e, wc):
                dm = plsc.load_gather(
                    pv[b], [jnp.zeros((16,), jnp.int32) + e,
                            jnp.zeros((16,), jnp.int32) + 3])
                for j in range(_D // 16):
                    dsl = pl.ds(j * 16, 16)
                    av[b][e, dsl] = (av[b][e, dsl] + bv[b][e, dsl]
                                     + dm * wc[j])
                return wc

            lax.fori_loop(0, _CG, ebody, wtup)

        def writeback(i, b):
            base = wid * ew + i * _CG
            pltpu.async_copy(av[b], u_hbm.at[pl.ds(base, _CG)], sw[b])
            pltpu.async_copy(pv[b], pay_hbm.at[pl.ds(base, _CG)], sw[b])

        def drain_w(i, b):
            base = wid * ew + i * _CG
            pltpu.make_async_copy(av[b], u_hbm.at[pl.ds(base, _CG)],
                                  sw[b]).wait()
            pltpu.make_async_copy(pv[b], pay_hbm.at[pl.ds(base, _CG)],
                                  sw[b]).wait()

        def step(i, b):
            cpa, cpb = load_and_gather(i, b)
            payload(b)
            cpa.wait()
            cpb.wait()
            fold(b)
            writeback(i, b)

        for b in range(2):
            step(jnp.int32(b), b)

        def outer(g, carry):
            for b in range(2):
                i = 2 * g + b
                drain_w(i, b)
                step(i, b)
            return carry

        npair = nchunk // 2
        lax.fori_loop(1, npair, outer, 0)
        if nchunk % 2:
            i = jnp.int32(nchunk - 1)
            drain_w(i, 0)
            step(i, 0)
            drain_w(jnp.int32(nchunk - 2), 1)
            drain_w(i, 0)
        else:
            for b in range(2):
                drain_w(jnp.int32(nchunk - 2 + b), b)

    return gather_kernel(ta, tb, coordflat, wrow1d, src3, dst3)


def _sc_scatter(mh, mx, dst3, zh, zx, n_nodes):
    nchunk = dst3.shape[0] // _NW
    ew = nchunk * _CS
    rows_per_tile = n_nodes // _NS
    mesh = plsc.VectorSubcoreMesh(core_axis_name="c", subcore_axis_name="s")

    @functools.partial(
        pl.kernel,
        out_type=(jax.ShapeDtypeStruct((_NC, n_nodes, _D), jnp.float32),
                  jax.ShapeDtypeStruct((_NC, n_nodes, 16), jnp.float32)),
        mesh=mesh,
        name="sc_scatter",
        scratch_types=[
            pltpu.VMEM((_CS,), jnp.int32),
            pltpu.VMEM((_CS,), jnp.int32),
            pltpu.VMEM((_CS, _D), jnp.float32),
            pltpu.VMEM((_CS, _D), jnp.float32),
            pltpu.VMEM((_CS, 16), jnp.float32),
            pltpu.VMEM((_CS, 16), jnp.float32),
            pltpu.VMEM_SHARED((n_nodes, _D), jnp.float32),
            pltpu.VMEM_SHARED((n_nodes, 16), jnp.float32),
            pltpu.SemaphoreType.DMA,
            pltpu.SemaphoreType.DMA,
            pltpu.SemaphoreType.DMA,
            pltpu.SemaphoreType.DMA,
        ],
        compiler_params=pltpu.CompilerParams(use_tc_tiling_on_sc=False),
    )
    def scatter_kernel(mh_hbm, mx_hbm, dst_hbm, zh_hbm, zx_hbm,
                       oh_hbm, ox_hbm, i0, i1, mh0, mh1, mx0, mx1,
                       acch_sh, accx_sh, sl0, sl1, ss0, ss1):
        cid = lax.axis_index("c")
        sid = lax.axis_index("s")
        wid = sid * _NC + cid
        iv = (i0, i1)
        mhv = (mh0, mh1)
        mxv = (mx0, mx1)
        sl = (sl0, sl1)
        ss = (ss0, ss1)
        r0 = sid * rows_per_tile
        rows = pl.ds(r0, rows_per_tile)
        pltpu.sync_copy(zh_hbm.at[cid, rows], acch_sh.at[rows])
        pltpu.sync_copy(zx_hbm.at[cid, rows], accx_sh.at[rows])
        plsc.subcore_barrier()

        def load(i, b):
            base = wid * ew + i * _CS
            pltpu.async_copy(dst_hbm.at[wid * nchunk + i], iv[b], sl[b])
            pltpu.async_copy(mh_hbm.at[pl.ds(base, _CS)], mhv[b], sl[b])
            pltpu.async_copy(mx_hbm.at[pl.ds(base, _CS)], mxv[b], sl[b])

        def drain_l(i, b):
            base = wid * ew + i * _CS
            pltpu.make_async_copy(dst_hbm.at[wid * nchunk + i], iv[b],
                                  sl[b]).wait()
            pltpu.make_async_copy(mh_hbm.at[pl.ds(base, _CS)], mhv[b],
                                  sl[b]).wait()
            pltpu.make_async_copy(mx_hbm.at[pl.ds(base, _CS)], mxv[b],
                                  sl[b]).wait()

        def add(b):
            pltpu.async_copy(mhv[b], acch_sh.at[iv[b]], ss[b], add=True)
            pltpu.async_copy(mxv[b], accx_sh.at[iv[b]], ss[b], add=True)

        def drain_s(b):
            pltpu.make_async_copy(mhv[b], acch_sh.at[iv[b]], ss[b]).wait()
            pltpu.make_async_copy(mxv[b], accx_sh.at[iv[b]], ss[b]).wait()

        for b in range(2):
            load(jnp.int32(b), b)
        for b in range(2):
            drain_l(jnp.int32(b), b)
            add(b)

        def outer(g, carry):
            for b in range(2):
                i = 2 * g + b
                drain_s(b)
                load(i, b)
                drain_l(i, b)
                add(b)
            return carry

        lax.fori_loop(1, nchunk // 2, outer, 0)
        for b in range(2):
            drain_s(b)
        plsc.subcore_barrier()
        pltpu.sync_copy(acch_sh.at[rows], oh_hbm.at[cid, rows])
        pltpu.sync_copy(accx_sh.at[rows], ox_hbm.at[cid, rows])

    return scatter_kernel(mh, mx, dst3, zh, zx)


def _tc_prep(feat, coordpad, we1a, we1b, wrow, be1):
    n = feat.shape[0]
    bn = 1000

    def body(f_ref, c_ref, wa_ref, wb_ref, wr_ref, b1_ref, ta_ref, tb_ref):
        f = f_ref[...]
        c = c_ref[...]
        sq = jnp.sum(c * c, axis=1, keepdims=True)
        sqw = sq * wr_ref[...]
        ta_ref[...] = jnp.dot(f, wa_ref[...],
                              preferred_element_type=jnp.float32) + sqw
        tb_ref[...] = jnp.dot(f, wb_ref[...],
                              preferred_element_type=jnp.float32) + sqw + b1_ref[...]

    return pl.pallas_call(
        body,
        grid=(n // bn,),
        in_specs=[
            pl.BlockSpec((bn, _D), lambda i: (i, 0)),
            pl.BlockSpec((bn, 16), lambda i: (i, 0)),
            pl.BlockSpec((_D, _D), lambda i: (0, 0)),
            pl.BlockSpec((_D, _D), lambda i: (0, 0)),
            pl.BlockSpec((1, _D), lambda i: (0, 0)),
            pl.BlockSpec((1, _D), lambda i: (0, 0)),
        ],
        out_specs=[
            pl.BlockSpec((bn, _D), lambda i: (i, 0)),
            pl.BlockSpec((bn, _D), lambda i: (i, 0)),
        ],
        out_shape=[jax.ShapeDtypeStruct((n, _D), jnp.float32)] * 2,
    )(feat, coordpad, we1a, we1b, wrow, be1)


def _tc_edge(uu, pay, we2, be2, wc1, bc1, wc2rep, bc2rep):
    e_total = uu.shape[0]
    be = 1600

    def body(u_ref, pay_ref, w2_ref, be2_ref,
             wc1_ref, bc1_ref, wc2_ref, bc2_ref, mh_ref, mx_ref):
        p = pay_ref[...]
        u = u_ref[...]
        h1 = u * jax.nn.sigmoid(u)
        h2 = jnp.dot(h1, w2_ref[...],
                     preferred_element_type=jnp.float32) + be2_ref[...]
        h2 = h2 * jax.nn.sigmoid(h2)
        t = jnp.dot(h2, wc1_ref[...],
                    preferred_element_type=jnp.float32) + bc1_ref[...]
        t = t * jax.nn.sigmoid(t)
        cc = jnp.dot(t, wc2_ref[...],
                     preferred_element_type=jnp.float32) + bc2_ref[...]
        mh_ref[...] = h2
        mx_ref[...] = p * cc[:, :16]

    return pl.pallas_call(
        body,
        grid=(e_total // be,),
        in_specs=[
            pl.BlockSpec((be, _D), lambda i: (i, 0)),
            pl.BlockSpec((be, 16), lambda i: (i, 0)),
            pl.BlockSpec((_D, _D), lambda i: (0, 0)),
            pl.BlockSpec((1, _D), lambda i: (0, 0)),
            pl.BlockSpec((_D, _D), lambda i: (0, 0)),
            pl.BlockSpec((1, _D), lambda i: (0, 0)),
            pl.BlockSpec((_D, _D), lambda i: (0, 0)),
            pl.BlockSpec((1, _D), lambda i: (0, 0)),
        ],
        out_specs=[
            pl.BlockSpec((be, _D), lambda i: (i, 0)),
            pl.BlockSpec((be, 16), lambda i: (i, 0)),
        ],
        out_shape=[jax.ShapeDtypeStruct((e_total, _D), jnp.float32),
                   jax.ShapeDtypeStruct((e_total, 16), jnp.float32)],
    )(uu, pay, we2, be2, wc1, bc1, wc2rep, bc2rep)


def _tc_node(feat, coordpad, ph0, ph1, px0, px1, wn1a, wn1b, bn1, wn2, bn2):
    n = feat.shape[0]
    bn = 1000

    def body(f_ref, c_ref, ph0_ref, ph1_ref, px0_ref, px1_ref,
             wa_ref, wb_ref, b1_ref, w2_ref, b2_ref, x_ref, h_ref):
        hagg = ph0_ref[...] + ph1_ref[...]
        x_ref[...] = c_ref[...] + px0_ref[...] + px1_ref[...]
        u = (jnp.dot(f_ref[...], wa_ref[...],
                     preferred_element_type=jnp.float32)
             + jnp.dot(hagg, wb_ref[...],
                       preferred_element_type=jnp.float32)
             + b1_ref[...])
        h1 = u * jax.nn.sigmoid(u)
        h_ref[...] = jnp.dot(h1, w2_ref[...],
                             preferred_element_type=jnp.float32) + b2_ref[...]

    return pl.pallas_call(
        body,
        grid=(n // bn,),
        in_specs=[
            pl.BlockSpec((bn, _D), lambda i: (i, 0)),
            pl.BlockSpec((bn, 16), lambda i: (i, 0)),
            pl.BlockSpec((bn, _D), lambda i: (i, 0)),
            pl.BlockSpec((bn, _D), lambda i: (i, 0)),
            pl.BlockSpec((bn, 16), lambda i: (i, 0)),
            pl.BlockSpec((bn, 16), lambda i: (i, 0)),
            pl.BlockSpec((_D, _D), lambda i: (0, 0)),
            pl.BlockSpec((_D, _D), lambda i: (0, 0)),
            pl.BlockSpec((1, _D), lambda i: (0, 0)),
            pl.BlockSpec((_D, _D), lambda i: (0, 0)),
            pl.BlockSpec((1, _D), lambda i: (0, 0)),
        ],
        out_specs=[
            pl.BlockSpec((bn, 16), lambda i: (i, 0)),
            pl.BlockSpec((bn, _D), lambda i: (i, 0)),
        ],
        out_shape=[jax.ShapeDtypeStruct((n, 16), jnp.float32),
                   jax.ShapeDtypeStruct((n, _D), jnp.float32)],
    )(feat, coordpad, ph0, ph1, px0, px1, wn1a, wn1b, bn1, wn2, bn2)


def kernel(feat, coordinate, edge_index, We1, be1, We2, be2,
           Wc1, bc1, Wc2, bc2, Wn1, bn1, Wn2, bn2):
    n = feat.shape[0]
    e_total = edge_index.shape[1]
    src = edge_index[0].astype(jnp.int32)
    dst = edge_index[1].astype(jnp.int32)
    cuts = [0, 3 * e_total // 5, e_total]
    portions = list(zip(cuts[:-1], cuts[1:]))

    coordpad = jnp.pad(coordinate, ((0, 0), (0, 16 - coordinate.shape[1])))
    npad = -n % 256
    coordflat = jnp.pad(coordinate,
                        ((0, npad), (0, 1))).reshape((n + npad) // 32, _D)

    wrow = We1[2 * _D].reshape(1, _D)

    ta, tb = _tc_prep(feat, coordpad, We1[:_D], We1[_D:2 * _D], wrow,
                      be1.reshape(1, _D))

    msgs = []
    for lo, hi in portions:
        uu, pay = _sc_gather(ta, tb, coordflat, wrow.reshape(_D),
                             src[lo:hi].reshape(-1, _CG),
                             dst[lo:hi].reshape(-1, _CG), hi - lo)
        mh, mx = _tc_edge(uu, pay, We2, be2.reshape(1, _D), Wc1,
                          bc1.reshape(1, _D), jnp.tile(Wc2, (1, _D)),
                          jnp.tile(bc2, (_D,)).reshape(1, _D))
        msgs.append((mh, mx))

    ph = jnp.zeros((_NC, n, _D), jnp.float32)
    px = jnp.zeros((_NC, n, 16), jnp.float32)
    for (lo, hi), (mh, mx) in zip(portions, msgs):
        ph, px = _sc_scatter(mh, mx, dst[lo:hi].reshape(-1, _CS),
                             ph, px, n)

    x_pad, h_new = _tc_node(feat, coordpad, ph[0], ph[1], px[0], px[1],
                            Wn1[:_D], Wn1[_D:], bn1.reshape(1, _D),
                            Wn2, bn2.reshape(1, _D))
    return (x_pad[:, :coordinate.shape[1]], h_new)


# speedup vs baseline: 1.2647x; 1.2647x over previous
import functools

import jax
import jax.numpy as jnp
from jax import lax
from jax.experimental import pallas as pl
from jax.experimental.pallas import tpu as pltpu
from jax.experimental.pallas import tpu_sc as plsc

_NC = 2
_NS = 16
_NW = _NC * _NS

_D = 128
_CG = 80
_CS = 125


def _sc_gather(ta, tb, coordflat, src3, dst3, e_total):
    nchunk = src3.shape[0] // _NW
    ew = nchunk * _CG
    ncrows = coordflat.shape[0]
    mesh = plsc.VectorSubcoreMesh(core_axis_name="c", subcore_axis_name="s")

    @functools.partial(
        pl.kernel,
        out_type=(jax.ShapeDtypeStruct((e_total, _D), jnp.float32),
                  jax.ShapeDtypeStruct((e_total, _D), jnp.float32),
                  jax.ShapeDtypeStruct((e_total, 16), jnp.float32)),
        mesh=mesh,
        scratch_types=[
            pltpu.VMEM((_CG,), jnp.int32),
            pltpu.VMEM((_CG,), jnp.int32),
            pltpu.VMEM((_CG,), jnp.int32),
            pltpu.VMEM((_CG,), jnp.int32),
            pltpu.VMEM((_CG, _D), jnp.float32),
            pltpu.VMEM((_CG, _D), jnp.float32),
            pltpu.VMEM((_CG, _D), jnp.float32),
            pltpu.VMEM((_CG, _D), jnp.float32),
            pltpu.VMEM((_CG, 16), jnp.float32),
            pltpu.VMEM((_CG, 16), jnp.float32),
            pltpu.VMEM((ncrows, _D), jnp.float32),
            pltpu.SemaphoreType.DMA,
            pltpu.SemaphoreType.DMA,
            pltpu.SemaphoreType.DMA,
            pltpu.SemaphoreType.DMA,
        ],
        compiler_params=pltpu.CompilerParams(needs_layout_passes=False),
    )
    def gather_kernel(ta_hbm, tb_hbm, cf_hbm, src_hbm, dst_hbm,
                      g1_hbm, g2_hbm, pay_hbm,
                      si0, si1, di0, di1, a0, a1, b0, b1, p0, p1,
                      coord_v, sg0, sg1, sw0, sw1):
        wid = lax.axis_index("s") * _NC + lax.axis_index("c")
        si = (si0, si1)
        di = (di0, di1)
        av = (a0, a1)
        bv = (b0, b1)
        pv = (p0, p1)
        sg = (sg0, sg1)
        sw = (sw0, sw1)
        pltpu.sync_copy(cf_hbm, coord_v)

        def load_and_gather(i, b):
            pltpu.sync_copy(src_hbm.at[wid * nchunk + i], si[b])
            pltpu.sync_copy(dst_hbm.at[wid * nchunk + i], di[b])
            return (pltpu.async_copy(ta_hbm.at[si[b]], av[b], sg[b]),
                    pltpu.async_copy(tb_hbm.at[di[b]], bv[b], sg[b]))

        def payload(b):
            one = jnp.ones((16,), jnp.int32)
            for g in range(_CG // 16):
                sl = pl.ds(g * 16, 16)
                s = si[b][sl]
                d = di[b][sl]
                rs = lax.shift_right_logical(s, 5)
                cs = lax.shift_left(lax.bitwise_and(s, 31), 2)
                rd = lax.shift_right_logical(d, 5)
                cd = lax.shift_left(lax.bitwise_and(d, 31), 2)
                xs0 = plsc.load_gather(coord_v, [rs, cs])
                xs1 = plsc.load_gather(coord_v, [rs, cs + one])
                xs2 = plsc.load_gather(coord_v, [rs, cs + 2 * one])
                xd0 = plsc.load_gather(coord_v, [rd, cd])
                xd1 = plsc.load_gather(coord_v, [rd, cd + one])
                xd2 = plsc.load_gather(coord_v, [rd, cd + 2 * one])
                erows = lax.iota(jnp.int32, 16) + g * 16
                plsc.store_scatter(pv[b], [erows, 0 * one], xs0 - xd0)
                plsc.store_scatter(pv[b], [erows, 1 * one], xs1 - xd1)
                plsc.store_scatter(pv[b], [erows, 2 * one], xs2 - xd2)
                dotm2 = (xs0 * xd0 + xs1 * xd1 + xs2 * xd2) * (-2.0)
                plsc.store_scatter(pv[b], [erows, 3 * one], dotm2)

        def writeback(i, b):
            base = wid * ew + i * _CG
            pltpu.async_copy(av[b], g1_hbm.at[pl.ds(base, _CG)], sw[b])
            pltpu.async_copy(bv[b], g2_hbm.at[pl.ds(base, _CG)], sw[b])
            pltpu.async_copy(pv[b], pay_hbm.at[pl.ds(base, _CG)], sw[b])

        def drain_w(i, b):
            base = wid * ew + i * _CG
            pltpu.make_async_copy(av[b], g1_hbm.at[pl.ds(base, _CG)],
                                  sw[b]).wait()
            pltpu.make_async_copy(bv[b], g2_hbm.at[pl.ds(base, _CG)],
                                  sw[b]).wait()
            pltpu.make_async_copy(pv[b], pay_hbm.at[pl.ds(base, _CG)],
                                  sw[b]).wait()

        def step(i, b):
            cpa, cpb = load_and_gather(i, b)
            payload(b)
            cpa.wait()
            cpb.wait()
            writeback(i, b)

        for b in range(2):
            step(jnp.int32(b), b)

        def outer(g, carry):
            for b in range(2):
                i = 2 * g + b
                drain_w(i, b)
                step(i, b)
            return carry

        npair = nchunk // 2
        lax.fori_loop(1, npair, outer, 0)
        if nchunk % 2:
            i = jnp.int32(nchunk - 1)
            drain_w(i, 0)
            step(i, 0)
            drain_w(jnp.int32(nchunk - 2), 1)
            drain_w(i, 0)
        else:
            for b in range(2):
                drain_w(jnp.int32(nchunk - 2 + b), b)

    return gather_kernel(ta, tb, coordflat, src3, dst3)


def _sc_scatter(mh, mx, dst3, zh, zx, n_nodes):
    nchunk = dst3.shape[0] // _NW
    ew = nchunk * _CS
    rows_per_tile = n_nodes // _NS
    mesh = plsc.VectorSubcoreMesh(core_axis_name="c", subcore_axis_name="s")

    @functools.partial(
        pl.kernel,
        out_type=(jax.ShapeDtypeStruct((_NC, n_nodes, _D), jnp.float32),
                  jax.ShapeDtypeStruct((_NC, n_nodes, 16), jnp.float32)),
        mesh=mesh,
        name="sc_scatter",
        scratch_types=[
            pltpu.VMEM((_CS,), jnp.int32),
            pltpu.VMEM((_CS,), jnp.int32),
            pltpu.VMEM((_CS, _D), jnp.float32),
            pltpu.VMEM((_CS, _D), jnp.float32),
            pltpu.VMEM((_CS, 16), jnp.float32),
            pltpu.VMEM((_CS, 16), jnp.float32),
            pltpu.VMEM_SHARED((n_nodes, _D), jnp.float32),
            pltpu.VMEM_SHARED((n_nodes, 16), jnp.float32),
            pltpu.SemaphoreType.DMA,
            pltpu.SemaphoreType.DMA,
            pltpu.SemaphoreType.DMA,
            pltpu.SemaphoreType.DMA,
        ],
        compiler_params=pltpu.CompilerParams(use_tc_tiling_on_sc=False),
    )
    def scatter_kernel(mh_hbm, mx_hbm, dst_hbm, zh_hbm, zx_hbm,
                       oh_hbm, ox_hbm, i0, i1, mh0, mh1, mx0, mx1,
                       acch_sh, accx_sh, sl0, sl1, ss0, ss1):
        cid = lax.axis_index("c")
        sid = lax.axis_index("s")
        wid = sid * _NC + cid
        iv = (i0, i1)
        mhv = (mh0, mh1)
        mxv = (mx0, mx1)
        sl = (sl0, sl1)
        ss = (ss0, ss1)
        r0 = sid * rows_per_tile
        rows = pl.ds(r0, rows_per_tile)
        pltpu.sync_copy(zh_hbm.at[cid, rows], acch_sh.at[rows])
        pltpu.sync_copy(zx_hbm.at[cid, rows], accx_sh.at[rows])
        plsc.subcore_barrier()

        def load(i, b):
            base = wid * ew + i * _CS
            pltpu.async_copy(dst_hbm.at[wid * nchunk + i], iv[b], sl[b])
            pltpu.async_copy(mh_hbm.at[pl.ds(base, _CS)], mhv[b], sl[b])
            pltpu.async_copy(mx_hbm.at[pl.ds(base, _CS)], mxv[b], sl[b])

        def drain_l(i, b):
            base = wid * ew + i * _CS
            pltpu.make_async_copy(dst_hbm.at[wid * nchunk + i], iv[b],
                                  sl[b]).wait()
            pltpu.make_async_copy(mh_hbm.at[pl.ds(base, _CS)], mhv[b],
                                  sl[b]).wait()
            pltpu.make_async_copy(mx_hbm.at[pl.ds(base, _CS)], mxv[b],
                                  sl[b]).wait()

        def add(b):
            pltpu.async_copy(mhv[b], acch_sh.at[iv[b]], ss[b], add=True)
            pltpu.async_copy(mxv[b], accx_sh.at[iv[b]], ss[b], add=True)

        def drain_s(b):
            pltpu.make_async_copy(mhv[b], acch_sh.at[iv[b]], ss[b]).wait()
            pltpu.make_async_copy(mxv[b], accx_sh.at[iv[b]], ss[b]).wait()

        for b in range(2):
            load(jnp.int32(b), b)
        for b in range(2):
            drain_l(jnp.int32(b), b)
            add(b)

        def outer(g, carry):
            for b in range(2):
                i = 2 * g + b
                drain_s(b)
                load(i, b)
                drain_l(i, b)
                add(b)
            return carry

        lax.fori_loop(1, nchunk // 2, outer, 0)
        for b in range(2):
            drain_s(b)
        plsc.subcore_barrier()
        pltpu.sync_copy(acch_sh.at[rows], oh_hbm.at[cid, rows])
        pltpu.sync_copy(accx_sh.at[rows], ox_hbm.at[cid, rows])

    return scatter_kernel(mh, mx, dst3, zh, zx)


def _tc_prep(feat, coordpad, we1a, we1b, wrow, be1):
    n = feat.shape[0]
    bn = 1000

    def body(f_ref, c_ref, wa_ref, wb_ref, wr_ref, b1_ref, ta_ref, tb_ref):
        f = f_ref[...]
        c = c_ref[...]
        sq = jnp.sum(c * c, axis=1, keepdims=True)
        sqw = sq * wr_ref[...]
        ta_ref[...] = jnp.dot(f, wa_ref[...],
                              preferred_element_type=jnp.float32) + sqw
        tb_ref[...] = jnp.dot(f, wb_ref[...],
                              preferred_element_type=jnp.float32) + sqw + b1_ref[...]

    return pl.pallas_call(
        body,
        grid=(n // bn,),
        in_specs=[
            pl.BlockSpec((bn, _D), lambda i: (i, 0)),
            pl.BlockSpec((bn, 16), lambda i: (i, 0)),
            pl.BlockSpec((_D, _D), lambda i: (0, 0)),
            pl.BlockSpec((_D, _D), lambda i: (0, 0)),
            pl.BlockSpec((1, _D), lambda i: (0, 0)),
            pl.BlockSpec((1, _D), lambda i: (0, 0)),
        ],
        out_specs=[
            pl.BlockSpec((bn, _D), lambda i: (i, 0)),
            pl.BlockSpec((bn, _D), lambda i: (i, 0)),
        ],
        out_shape=[jax.ShapeDtypeStruct((n, _D), jnp.float32)] * 2,
    )(feat, coordpad, we1a, we1b, wrow, be1)


def _tc_edge(g1, g2, pay, wrow, we2, be2, wc1, bc1, wc2rep, bc2rep):
    e_total = g1.shape[0]
    be = 1600

    def body(g1_ref, g2_ref, pay_ref, wrow_ref, w2_ref, be2_ref,
             wc1_ref, bc1_ref, wc2_ref, bc2_ref, mh_ref, mx_ref):
        p = pay_ref[...]
        u = g1_ref[...] + g2_ref[...] + p[:, 3:4] * wrow_ref[...]
        h1 = u * jax.nn.sigmoid(u)
        h2 = jnp.dot(h1, w2_ref[...],
                     preferred_element_type=jnp.float32) + be2_ref[...]
        h2 = h2 * jax.nn.sigmoid(h2)
        t = jnp.dot(h2, wc1_ref[...],
                    preferred_element_type=jnp.float32) + bc1_ref[...]
        t = t * jax.nn.sigmoid(t)
        cc = jnp.dot(t, wc2_ref[...],
                     preferred_element_type=jnp.float32) + bc2_ref[...]
        mh_ref[...] = h2
        mx_ref[...] = p * cc[:, :16]

    return pl.pallas_call(
        body,
        grid=(e_total // be,),
        in_specs=[
            pl.BlockSpec((be, _D), lambda i: (i, 0)),
            pl.BlockSpec((be, _D), lambda i: (i, 0)),
            pl.BlockSpec((be, 16), lambda i: (i, 0)),
            pl.BlockSpec((1, _D), lambda i: (0, 0)),
            pl.BlockSpec((_D, _D), lambda i: (0, 0)),
            pl.BlockSpec((1, _D), lambda i: (0, 0)),
            pl.BlockSpec((_D, _D), lambda i: (0, 0)),
            pl.BlockSpec((1, _D), lambda i: (0, 0)),
            pl.BlockSpec((_D, _D), lambda i: (0, 0)),
            pl.BlockSpec((1, _D), lambda i: (0, 0)),
        ],
        out_specs=[
            pl.BlockSpec((be, _D), lambda i: (i, 0)),
            pl.BlockSpec((be, 16), lambda i: (i, 0)),
        ],
        out_shape=[jax.ShapeDtypeStruct((e_total, _D), jnp.float32),
                   jax.ShapeDtypeStruct((e_total, 16), jnp.float32)],
    )(g1, g2, pay, wrow, we2, be2, wc1, bc1, wc2rep, bc2rep)


def _tc_node(feat, coordpad, ph0, ph1, px0, px1, wn1a, wn1b, bn1, wn2, bn2):
    n = feat.shape[0]
    bn = 1000

    def body(f_ref, c_ref, ph0_ref, ph1_ref, px0_ref, px1_ref,
             wa_ref, wb_ref, b1_ref, w2_ref, b2_ref, x_ref, h_ref):
        hagg = ph0_ref[...] + ph1_ref[...]
        x_ref[...] = c_ref[...] + px0_ref[...] + px1_ref[...]
        u = (jnp.dot(f_ref[...], wa_ref[...],
                     preferred_element_type=jnp.float32)
             + jnp.dot(hagg, wb_ref[...],
                       preferred_element_type=jnp.float32)
             + b1_ref[...])
        h1 = u * jax.nn.sigmoid(u)
        h_ref[...] = jnp.dot(h1, w2_ref[...],
                             preferred_element_type=jnp.float32) + b2_ref[...]

    return pl.pallas_call(
        body,
        grid=(n // bn,),
        in_specs=[
            pl.BlockSpec((bn, _D), lambda i: (i, 0)),
            pl.BlockSpec((bn, 16), lambda i: (i, 0)),
            pl.BlockSpec((bn, _D), lambda i: (i, 0)),
            pl.BlockSpec((bn, _D), lambda i: (i, 0)),
            pl.BlockSpec((bn, 16), lambda i: (i, 0)),
            pl.BlockSpec((bn, 16), lambda i: (i, 0)),
            pl.BlockSpec((_D, _D), lambda i: (0, 0)),
            pl.BlockSpec((_D, _D), lambda i: (0, 0)),
            pl.BlockSpec((1, _D), lambda i: (0, 0)),
            pl.BlockSpec((_D, _D), lambda i: (0, 0)),
            pl.BlockSpec((1, _D), lambda i: (0, 0)),
        ],
        out_specs=[
            pl.BlockSpec((bn, 16), lambda i: (i, 0)),
            pl.BlockSpec((bn, _D), lambda i: (i, 0)),
        ],
        out_shape=[jax.ShapeDtypeStruct((n, 16), jnp.float32),
                   jax.ShapeDtypeStruct((n, _D), jnp.float32)],
    )(feat, coordpad, ph0, ph1, px0, px1, wn1a, wn1b, bn1, wn2, bn2)


def kernel(feat, coordinate, edge_index, We1, be1, We2, be2,
           Wc1, bc1, Wc2, bc2, Wn1, bn1, Wn2, bn2):
    n = feat.shape[0]
    e_total = edge_index.shape[1]
    src = edge_index[0].astype(jnp.int32)
    dst = edge_index[1].astype(jnp.int32)
    cuts = [0, 3 * e_total // 5, e_total]
    portions = list(zip(cuts[:-1], cuts[1:]))

    coordpad = jnp.pad(coordinate, ((0, 0), (0, 16 - coordinate.shape[1])))
    npad = -n % 256
    coordflat = jnp.pad(coordinate,
                        ((0, npad), (0, 1))).reshape((n + npad) // 32, _D)

    wrow = We1[2 * _D].reshape(1, _D)

    ta, tb = _tc_prep(feat, coordpad, We1[:_D], We1[_D:2 * _D], wrow,
                      be1.reshape(1, _D))

    msgs = []
    for lo, hi in portions:
        g1, g2, pay = _sc_gather(ta, tb, coordflat,
                                 src[lo:hi].reshape(-1, _CG),
                                 dst[lo:hi].reshape(-1, _CG), hi - lo)
        mh, mx = _tc_edge(g1, g2, pay, wrow, We2, be2.reshape(1, _D), Wc1,
                          bc1.reshape(1, _D), jnp.tile(Wc2, (1, _D)),
                          jnp.tile(bc2, (_D,)).reshape(1, _D))
        msgs.append((mh, mx))

    ph = jnp.zeros((_NC, n, _D), jnp.float32)
    px = jnp.zeros((_NC, n, 16), jnp.float32)
    for (lo, hi), (mh, mx) in zip(portions, msgs):
        ph, px = _sc_scatter(mh, mx, dst[lo:hi].reshape(-1, _CS),
                             ph, px, n)

    x_pad, h_new = _tc_node(feat, coordpad, ph[0], ph[1], px[0], px[1],
                            Wn1[:_D], Wn1[_D:], bn1.reshape(1, _D),
                            Wn2, bn2.reshape(1, _D))
    return (x_pad[:, :coordinate.shape[1]], h_new)


# speedup vs baseline: 1.3289x; 1.0507x over previous
import functools

import jax
import jax.numpy as jnp
from jax import lax
from jax.experimental import pallas as pl
from jax.experimental.pallas import tpu as pltpu
from jax.experimental.pallas import tpu_sc as plsc

_NC = 2
_NS = 16
_NW = _NC * _NS

_D = 128
_CG = 80
_CS = 125


def _sc_gather(ta, tb, coordflat, src3, dst3, e_total):
    nchunk = src3.shape[0] // _NW
    ew = nchunk * _CG
    ncrows = coordflat.shape[0]
    mesh = plsc.VectorSubcoreMesh(core_axis_name="c", subcore_axis_name="s")

    @functools.partial(
        pl.kernel,
        out_type=(jax.ShapeDtypeStruct((e_total, _D), jnp.float32),
                  jax.ShapeDtypeStruct((e_total, _D), jnp.float32),
                  jax.ShapeDtypeStruct((e_total, 16), jnp.float32)),
        mesh=mesh,
        scratch_types=[
            pltpu.VMEM((_CG,), jnp.int32),
            pltpu.VMEM((_CG,), jnp.int32),
            pltpu.VMEM((_CG,), jnp.int32),
            pltpu.VMEM((_CG,), jnp.int32),
            pltpu.VMEM((_CG, _D), jnp.float32),
            pltpu.VMEM((_CG, _D), jnp.float32),
            pltpu.VMEM((_CG, _D), jnp.float32),
            pltpu.VMEM((_CG, _D), jnp.float32),
            pltpu.VMEM((_CG, 16), jnp.float32),
            pltpu.VMEM((_CG, 16), jnp.float32),
            pltpu.VMEM((ncrows, _D), jnp.float32),
            pltpu.SemaphoreType.DMA,
            pltpu.SemaphoreType.DMA,
            pltpu.SemaphoreType.DMA,
            pltpu.SemaphoreType.DMA,
        ],
        compiler_params=pltpu.CompilerParams(needs_layout_passes=False),
    )
    def gather_kernel(ta_hbm, tb_hbm, cf_hbm, src_hbm, dst_hbm,
                      g1_hbm, g2_hbm, pay_hbm,
                      si0, si1, di0, di1, a0, a1, b0, b1, p0, p1,
                      coord_v, sg0, sg1, sw0, sw1):
        wid = lax.axis_index("s") * _NC + lax.axis_index("c")
        si = (si0, si1)
        di = (di0, di1)
        av = (a0, a1)
        bv = (b0, b1)
        pv = (p0, p1)
        sg = (sg0, sg1)
        sw = (sw0, sw1)
        pltpu.sync_copy(cf_hbm, coord_v)

        def load_and_gather(i, b):
            pltpu.sync_copy(src_hbm.at[wid * nchunk + i], si[b])
            pltpu.sync_copy(dst_hbm.at[wid * nchunk + i], di[b])
            return (pltpu.async_copy(ta_hbm.at[si[b]], av[b], sg[b]),
                    pltpu.async_copy(tb_hbm.at[di[b]], bv[b], sg[b]))

        def payload(b):
            one = jnp.ones((16,), jnp.int32)
            for g in range(_CG // 16):
                sl = pl.ds(g * 16, 16)
                s = si[b][sl]
                d = di[b][sl]
                rs = lax.shift_right_logical(s, 5)
                cs = lax.shift_left(lax.bitwise_and(s, 31), 2)
                rd = lax.shift_right_logical(d, 5)
                cd = lax.shift_left(lax.bitwise_and(d, 31), 2)
                xs0 = plsc.load_gather(coord_v, [rs, cs])
                xs1 = plsc.load_gather(coord_v, [rs, cs + one])
                xs2 = plsc.load_gather(coord_v, [rs, cs + 2 * one])
                xd0 = plsc.load_gather(coord_v, [rd, cd])
                xd1 = plsc.load_gather(coord_v, [rd, cd + one])
                xd2 = plsc.load_gather(coord_v, [rd, cd + 2 * one])
                erows = lax.iota(jnp.int32, 16) + g * 16
                plsc.store_scatter(pv[b], [erows, 0 * one], xs0 - xd0)
                plsc.store_scatter(pv[b], [erows, 1 * one], xs1 - xd1)
                plsc.store_scatter(pv[b], [erows, 2 * one], xs2 - xd2)
                dotm2 = (xs0 * xd0 + xs1 * xd1 + xs2 * xd2) * (-2.0)
                plsc.store_scatter(pv[b], [erows, 3 * one], dotm2)

        def writeback(i, b):
            base = wid * ew + i * _CG
            pltpu.async_copy(av[b], g1_hbm.at[pl.ds(base, _CG)], sw[b])
            pltpu.async_copy(bv[b], g2_hbm.at[pl.ds(base, _CG)], sw[b])
            pltpu.async_copy(pv[b], pay_hbm.at[pl.ds(base, _CG)], sw[b])

        def drain_w(i, b):
            base = wid * ew + i * _CG
            pltpu.make_async_copy(av[b], g1_hbm.at[pl.ds(base, _CG)],
                                  sw[b]).wait()
            pltpu.make_async_copy(bv[b], g2_hbm.at[pl.ds(base, _CG)],
                                  sw[b]).wait()
            pltpu.make_async_copy(pv[b], pay_hbm.at[pl.ds(base, _CG)],
                                  sw[b]).wait()

        def step(i, b):
            cpa, cpb = load_and_gather(i, b)
            payload(b)
            cpa.wait()
            cpb.wait()
            writeback(i, b)

        for b in range(2):
            step(jnp.int32(b), b)

        def outer(g, carry):
            for b in range(2):
                i = 2 * g + b
                drain_w(i, b)
                step(i, b)
            return carry

        npair = nchunk // 2
        lax.fori_loop(1, npair, outer, 0)
        if nchunk % 2:
            i = jnp.int32(nchunk - 1)
            drain_w(i, 0)
            step(i, 0)
            drain_w(jnp.int32(nchunk - 2), 1)
            drain_w(i, 0)
        else:
            for b in range(2):
                drain_w(jnp.int32(nchunk - 2 + b), b)

    return gather_kernel(ta, tb, coordflat, src3, dst3)


def _sc_scatter(mh, mx, dst3, zh, zx, n_nodes):
    nchunk = dst3.shape[0] // _NW
    ew = nchunk * _CS
    rows_per_tile = n_nodes // _NS
    mesh = plsc.VectorSubcoreMesh(core_axis_name="c", subcore_axis_name="s")

    @functools.partial(
        pl.kernel,
        out_type=(jax.ShapeDtypeStruct((_NC, n_nodes, _D), jnp.float32),
                  jax.ShapeDtypeStruct((_NC, n_nodes, 16), jnp.float32)),
        mesh=mesh,
        name="sc_scatter",
        scratch_types=[
            pltpu.VMEM((_CS,), jnp.int32),
            pltpu.VMEM((_CS,), jnp.int32),
            pltpu.VMEM((_CS, _D), jnp.float32),
            pltpu.VMEM((_CS, _D), jnp.float32),
            pltpu.VMEM((_CS, 16), jnp.float32),
            pltpu.VMEM((_CS, 16), jnp.float32),
            pltpu.VMEM_SHARED((n_nodes, _D), jnp.float32),
            pltpu.VMEM_SHARED((n_nodes, 16), jnp.float32),
            pltpu.SemaphoreType.DMA,
            pltpu.SemaphoreType.DMA,
            pltpu.SemaphoreType.DMA,
            pltpu.SemaphoreType.DMA,
        ],
        compiler_params=pltpu.CompilerParams(use_tc_tiling_on_sc=False),
    )
    def scatter_kernel(mh_hbm, mx_hbm, dst_hbm, zh_hbm, zx_hbm,
                       oh_hbm, ox_hbm, i0, i1, mh0, mh1, mx0, mx1,
                       acch_sh, accx_sh, sl0, sl1, ss0, ss1):
        cid = lax.axis_index("c")
        sid = lax.axis_index("s")
        wid = sid * _NC + cid
        iv = (i0, i1)
        mhv = (mh0, mh1)
        mxv = (mx0, mx1)
        sl = (sl0, sl1)
        ss = (ss0, ss1)
        r0 = sid * rows_per_tile
        rows = pl.ds(r0, rows_per_tile)
        pltpu.sync_copy(zh_hbm.at[cid, rows], acch_sh.at[rows])
        pltpu.sync_copy(zx_hbm.at[cid, rows], accx_sh.at[rows])
        plsc.subcore_barrier()

        def load(i, b):
            base = wid * ew + i * _CS
            pltpu.async_copy(dst_hbm.at[wid * nchunk + i], iv[b], sl[b])
            pltpu.async_copy(mh_hbm.at[pl.ds(base, _CS)], mhv[b], sl[b])
            pltpu.async_copy(mx_hbm.at[pl.ds(base, _CS)], mxv[b], sl[b])

        def drain_l(i, b):
            base = wid * ew + i * _CS
            pltpu.make_async_copy(dst_hbm.at[wid * nchunk + i], iv[b],
                                  sl[b]).wait()
            pltpu.make_async_copy(mh_hbm.at[pl.ds(base, _CS)], mhv[b],
                                  sl[b]).wait()
            pltpu.make_async_copy(mx_hbm.at[pl.ds(base, _CS)], mxv[b],
                                  sl[b]).wait()

        def add(b):
            pltpu.async_copy(mhv[b], acch_sh.at[iv[b]], ss[b], add=True)
            pltpu.async_copy(mxv[b], accx_sh.at[iv[b]], ss[b], add=True)

        def drain_s(b):
            pltpu.make_async_copy(mhv[b], acch_sh.at[iv[b]], ss[b]).wait()
            pltpu.make_async_copy(mxv[b], accx_sh.at[iv[b]], ss[b]).wait()

        for b in range(2):
            load(jnp.int32(b), b)
        for b in range(2):
            drain_l(jnp.int32(b), b)
            add(b)

        def outer(g, carry):
            for b in range(2):
                i = 2 * g + b
                drain_s(b)
                load(i, b)
                drain_l(i, b)
                add(b)
            return carry

        lax.fori_loop(1, nchunk // 2, outer, 0)
        for b in range(2):
            drain_s(b)
        plsc.subcore_barrier()
        pltpu.sync_copy(acch_sh.at[rows], oh_hbm.at[cid, rows])
        pltpu.sync_copy(accx_sh.at[rows], ox_hbm.at[cid, rows])

    return scatter_kernel(mh, mx, dst3, zh, zx)


def _tc_prep(feat, coordpad, we1a, we1b, wrow, be1):
    n = feat.shape[0]
    bn = 1000

    def body(f_ref, c_ref, wa_ref, wb_ref, wr_ref, b1_ref, ta_ref, tb_ref):
        f = f_ref[...]
        c = c_ref[...]
        sq = jnp.sum(c * c, axis=1, keepdims=True)
        sqw = sq * wr_ref[...]
        ta_ref[...] = jnp.dot(f, wa_ref[...],
                              preferred_element_type=jnp.float32) + sqw
        tb_ref[...] = jnp.dot(f, wb_ref[...],
                              preferred_element_type=jnp.float32) + sqw + b1_ref[...]

    return pl.pallas_call(
        body,
        grid=(n // bn,),
        in_specs=[
            pl.BlockSpec((bn, _D), lambda i: (i, 0)),
            pl.BlockSpec((bn, 16), lambda i: (i, 0)),
            pl.BlockSpec((_D, _D), lambda i: (0, 0)),
            pl.BlockSpec((_D, _D), lambda i: (0, 0)),
            pl.BlockSpec((1, _D), lambda i: (0, 0)),
            pl.BlockSpec((1, _D), lambda i: (0, 0)),
        ],
        out_specs=[
            pl.BlockSpec((bn, _D), lambda i: (i, 0)),
            pl.BlockSpec((bn, _D), lambda i: (i, 0)),
        ],
        out_shape=[jax.ShapeDtypeStruct((n, _D), jnp.float32)] * 2,
    )(feat, coordpad, we1a, we1b, wrow, be1)


def _tc_edge(g1, g2, pay, wrow, we2, be2, wc1, bc1, wc2rep, bc2rep):
    e_total = g1.shape[0]
    be = 1600

    def body(g1_ref, g2_ref, pay_ref, wrow_ref, w2_ref, be2_ref,
             wc1_ref, bc1_ref, wc2_ref, bc2_ref, mh_ref, mx_ref):
        p = pay_ref[...]
        u = g1_ref[...] + g2_ref[...] + p[:, 3:4] * wrow_ref[...]
        h1 = u * jax.nn.sigmoid(u)
        h2 = jnp.dot(h1, w2_ref[...],
                     preferred_element_type=jnp.float32) + be2_ref[...]
        h2 = h2 * jax.nn.sigmoid(h2)
        t = jnp.dot(h2, wc1_ref[...],
                    preferred_element_type=jnp.float32) + bc1_ref[...]
        t = t * jax.nn.sigmoid(t)
        cc = jnp.dot(t, wc2_ref[...],
                     preferred_element_type=jnp.float32) + bc2_ref[...]
        mh_ref[...] = h2
        mx_ref[...] = p * cc[:, :16]

    return pl.pallas_call(
        body,
        grid=(e_total // be,),
        in_specs=[
            pl.BlockSpec((be, _D), lambda i: (i, 0)),
            pl.BlockSpec((be, _D), lambda i: (i, 0)),
            pl.BlockSpec((be, 16), lambda i: (i, 0)),
            pl.BlockSpec((1, _D), lambda i: (0, 0)),
            pl.BlockSpec((_D, _D), lambda i: (0, 0)),
            pl.BlockSpec((1, _D), lambda i: (0, 0)),
            pl.BlockSpec((_D, _D), lambda i: (0, 0)),
            pl.BlockSpec((1, _D), lambda i: (0, 0)),
            pl.BlockSpec((_D, _D), lambda i: (0, 0)),
            pl.BlockSpec((1, _D), lambda i: (0, 0)),
        ],
        out_specs=[
            pl.BlockSpec((be, _D), lambda i: (i, 0)),
            pl.BlockSpec((be, 16), lambda i: (i, 0)),
        ],
        out_shape=[jax.ShapeDtypeStruct((e_total, _D), jnp.float32),
                   jax.ShapeDtypeStruct((e_total, 16), jnp.float32)],
    )(g1, g2, pay, wrow, we2, be2, wc1, bc1, wc2rep, bc2rep)


def _tc_node(feat, coordpad, ph0, ph1, px0, px1, wn1a, wn1b, bn1, wn2, bn2):
    n = feat.shape[0]
    bn = 1000

    def body(f_ref, c_ref, ph0_ref, ph1_ref, px0_ref, px1_ref,
             wa_ref, wb_ref, b1_ref, w2_ref, b2_ref, x_ref, h_ref):
        hagg = ph0_ref[...] + ph1_ref[...]
        x_ref[...] = c_ref[...] + px0_ref[...] + px1_ref[...]
        u = (jnp.dot(f_ref[...], wa_ref[...],
                     preferred_element_type=jnp.float32)
             + jnp.dot(hagg, wb_ref[...],
                       preferred_element_type=jnp.float32)
             + b1_ref[...])
        h1 = u * jax.nn.sigmoid(u)
        h_ref[...] = jnp.dot(h1, w2_ref[...],
                             preferred_element_type=jnp.float32) + b2_ref[...]

    return pl.pallas_call(
        body,
        grid=(n // bn,),
        in_specs=[
            pl.BlockSpec((bn, _D), lambda i: (i, 0)),
            pl.BlockSpec((bn, 16), lambda i: (i, 0)),
            pl.BlockSpec((bn, _D), lambda i: (i, 0)),
            pl.BlockSpec((bn, _D), lambda i: (i, 0)),
            pl.BlockSpec((bn, 16), lambda i: (i, 0)),
            pl.BlockSpec((bn, 16), lambda i: (i, 0)),
            pl.BlockSpec((_D, _D), lambda i: (0, 0)),
            pl.BlockSpec((_D, _D), lambda i: (0, 0)),
            pl.BlockSpec((1, _D), lambda i: (0, 0)),
            pl.BlockSpec((_D, _D), lambda i: (0, 0)),
            pl.BlockSpec((1, _D), lambda i: (0, 0)),
        ],
        out_specs=[
            pl.BlockSpec((bn, 16), lambda i: (i, 0)),
            pl.BlockSpec((bn, _D), lambda i: (i, 0)),
        ],
        out_shape=[jax.ShapeDtypeStruct((n, 16), jnp.float32),
                   jax.ShapeDtypeStruct((n, _D), jnp.float32)],
    )(feat, coordpad, ph0, ph1, px0, px1, wn1a, wn1b, bn1, wn2, bn2)


def kernel(feat, coordinate, edge_index, We1, be1, We2, be2,
           Wc1, bc1, Wc2, bc2, Wn1, bn1, Wn2, bn2):
    n = feat.shape[0]
    e_total = edge_index.shape[1]
    src = edge_index[0].astype(jnp.int32)
    dst = edge_index[1].astype(jnp.int32)
    cuts = [0, 2 * e_total // 5, 4 * e_total // 5, e_total]
    portions = list(zip(cuts[:-1], cuts[1:]))

    coordpad = jnp.pad(coordinate, ((0, 0), (0, 16 - coordinate.shape[1])))
    npad = -n % 256
    coordflat = jnp.pad(coordinate,
                        ((0, npad), (0, 1))).reshape((n + npad) // 32, _D)

    wrow = We1[2 * _D].reshape(1, _D)

    ta, tb = _tc_prep(feat, coordpad, We1[:_D], We1[_D:2 * _D], wrow,
                      be1.reshape(1, _D))

    msgs = []
    for lo, hi in portions:
        g1, g2, pay = _sc_gather(ta, tb, coordflat,
                                 src[lo:hi].reshape(-1, _CG),
                                 dst[lo:hi].reshape(-1, _CG), hi - lo)
        mh, mx = _tc_edge(g1, g2, pay, wrow, We2, be2.reshape(1, _D), Wc1,
                          bc1.reshape(1, _D), jnp.tile(Wc2, (1, _D)),
                          jnp.tile(bc2, (_D,)).reshape(1, _D))
        msgs.append((mh, mx))

    ph = jnp.zeros((_NC, n, _D), jnp.float32)
    px = jnp.zeros((_NC, n, 16), jnp.float32)
    for (lo, hi), (mh, mx) in zip(portions, msgs):
        ph, px = _sc_scatter(mh, mx, dst[lo:hi].reshape(-1, _CS),
                             ph, px, n)

    x_pad, h_new = _tc_node(feat, coordpad, ph[0], ph[1], px[0], px[1],
                            Wn1[:_D], Wn1[_D:], bn1.reshape(1, _D),
                            Wn2, bn2.reshape(1, _D))
    return (x_pad[:, :coordinate.shape[1]], h_new)
